# baseline (speedup 1.0000x reference)
import functools

import jax
import jax.numpy as jnp
from jax import lax
from jax.experimental import pallas as pl
from jax.experimental.pallas import tpu as pltpu
from jax.experimental.pallas import tpu_sc as plsc

B, T, D = 1024, 200, 128
V = 100000

_NC, _NS = 2, 16
_NW = _NC * _NS
_B_PER_W = B // _NW


def _sc_gather(weight, idx):
    mesh = plsc.VectorSubcoreMesh(core_axis_name="c", subcore_axis_name="s")

    @functools.partial(
        pl.kernel,
        mesh=mesh,
        out_type=jax.ShapeDtypeStruct((B, D), jnp.float32),
        scratch_types=[
            pltpu.VMEM((_B_PER_W,), jnp.int32),
            pltpu.VMEM((_B_PER_W, D), jnp.float32),
            pltpu.SemaphoreType.DMA,
        ],
    )
    def gather_kernel(table_hbm, idx_hbm, out_hbm, idx_v, rows_v, sem):
        wid = lax.axis_index("s") * _NC + lax.axis_index("c")
        base = wid * _B_PER_W
        pltpu.sync_copy(idx_hbm.at[pl.ds(base, _B_PER_W)], idx_v)
        pltpu.async_copy(table_hbm.at[idx_v], rows_v, sem).wait()
        pltpu.sync_copy(rows_v, out_hbm.at[pl.ds(base, _B_PER_W)])

    return gather_kernel(weight, idx)


_BBLK = 32


def _scale_mul_body(rows_ref, x_ref, o_ref):
    s = 2.0 * jax.nn.sigmoid(rows_ref[...])
    o_ref[...] = x_ref[...] * s[:, None, :]


def _tc_scale_mul(rows, x):
    return pl.pallas_call(
        _scale_mul_body,
        grid=(B // _BBLK,),
        in_specs=[
            pl.BlockSpec((_BBLK, D), lambda i: (i, 0)),
            pl.BlockSpec((_BBLK, T, D), lambda i: (i, 0, 0)),
        ],
        out_specs=pl.BlockSpec((_BBLK, T, D), lambda i: (i, 0, 0)),
        out_shape=jax.ShapeDtypeStruct((B, T, D), jnp.float32),
    )(rows, x)


def kernel(x, spk_id, weight):
    idx = spk_id.reshape(-1)
    rows = _sc_gather(weight, idx)
    return _tc_scale_mul(rows, x)


# speedup vs baseline: 1.0291x; 1.0291x over previous
import functools

import jax
import jax.numpy as jnp
from jax import lax
from jax.experimental import pallas as pl
from jax.experimental.pallas import tpu as pltpu
from jax.experimental.pallas import tpu_sc as plsc

B, T, D = 1024, 200, 128
V = 100000

_NC, _NS = 2, 16
_NW = _NC * _NS
_B_PER_W = B // _NW


def _sc_gather(weight, idx):
    mesh = plsc.VectorSubcoreMesh(core_axis_name="c", subcore_axis_name="s")

    @functools.partial(
        pl.kernel,
        mesh=mesh,
        out_type=jax.ShapeDtypeStruct((B, D), jnp.float32),
        scratch_types=[
            pltpu.VMEM((_B_PER_W,), jnp.int32),
            pltpu.VMEM((_B_PER_W, D), jnp.float32),
            pltpu.SemaphoreType.DMA,
        ],
    )
    def gather_kernel(table_hbm, idx_hbm, out_hbm, idx_v, rows_v, sem):
        wid = lax.axis_index("s") * _NC + lax.axis_index("c")
        base = wid * _B_PER_W
        pltpu.sync_copy(idx_hbm.at[pl.ds(base, _B_PER_W)], idx_v)
        pltpu.async_copy(table_hbm.at[idx_v], rows_v, sem).wait()
        pltpu.sync_copy(rows_v, out_hbm.at[pl.ds(base, _B_PER_W)])

    return gather_kernel(weight, idx)


_BBLK = 64


def _scale_mul_body(rows_ref, x_ref, o_ref):
    s = 2.0 * jax.nn.sigmoid(rows_ref[...])
    o_ref[...] = x_ref[...] * s[:, None, :]


def _tc_scale_mul(rows, x):
    return pl.pallas_call(
        _scale_mul_body,
        grid=(B // _BBLK,),
        in_specs=[
            pl.BlockSpec((_BBLK, D), lambda i: (i, 0)),
            pl.BlockSpec((_BBLK, T, D), lambda i: (i, 0, 0)),
        ],
        out_specs=pl.BlockSpec((_BBLK, T, D), lambda i: (i, 0, 0)),
        out_shape=jax.ShapeDtypeStruct((B, T, D), jnp.float32),
    )(rows, x)


def kernel(x, spk_id, weight):
    idx = spk_id.reshape(-1)
    rows = _sc_gather(weight, idx)
    return _tc_scale_mul(rows, x)


# speedup vs baseline: 1.0350x; 1.0058x over previous
import functools

import jax
import jax.numpy as jnp
from jax import lax
from jax.experimental import pallas as pl
from jax.experimental.pallas import tpu as pltpu
from jax.experimental.pallas import tpu_sc as plsc

B, T, D = 1024, 200, 128
V = 100000

_NC, _NS = 2, 16
_NW = _NC * _NS
_B_PER_W = B // _NW


def _sc_gather(weight, idx):
    mesh = plsc.VectorSubcoreMesh(core_axis_name="c", subcore_axis_name="s")

    @functools.partial(
        pl.kernel,
        mesh=mesh,
        out_type=jax.ShapeDtypeStruct((B, D), jnp.float32),
        scratch_types=[
            pltpu.VMEM((_B_PER_W,), jnp.int32),
            pltpu.VMEM((_B_PER_W, D), jnp.float32),
            pltpu.SemaphoreType.DMA,
        ],
    )
    def gather_kernel(table_hbm, idx_hbm, out_hbm, idx_v, rows_v, sem):
        wid = lax.axis_index("s") * _NC + lax.axis_index("c")
        base = wid * _B_PER_W
        pltpu.sync_copy(idx_hbm.at[pl.ds(base, _B_PER_W)], idx_v)
        pltpu.async_copy(table_hbm.at[idx_v], rows_v, sem).wait()
        pltpu.sync_copy(rows_v, out_hbm.at[pl.ds(base, _B_PER_W)])

    return gather_kernel(weight, idx)


_BBLK = 128


def _scale_mul_body(rows_ref, x_ref, o_ref):
    s = 2.0 * jax.nn.sigmoid(rows_ref[...])
    o_ref[...] = x_ref[...] * s[:, None, :]


def _tc_scale_mul(rows, x):
    return pl.pallas_call(
        _scale_mul_body,
        grid=(B // _BBLK,),
        in_specs=[
            pl.BlockSpec((_BBLK, D), lambda i: (i, 0)),
            pl.BlockSpec((_BBLK, T, D), lambda i: (i, 0, 0)),
        ],
        out_specs=pl.BlockSpec((_BBLK, T, D), lambda i: (i, 0, 0)),
        out_shape=jax.ShapeDtypeStruct((B, T, D), jnp.float32),
    )(rows, x)


def kernel(x, spk_id, weight):
    idx = spk_id.reshape(-1)
    rows = _sc_gather(weight, idx)
    return _tc_scale_mul(rows, x)


# speedup vs baseline: 1.0382x; 1.0030x over previous
import functools

import jax
import jax.numpy as jnp
from jax import lax
from jax.experimental import pallas as pl
from jax.experimental.pallas import tpu as pltpu
from jax.experimental.pallas import tpu_sc as plsc

B, T, D = 1024, 200, 128
V = 100000

_NC, _NS = 2, 16
_NW = _NC * _NS
_B_PER_W = B // _NW


def _sc_gather(weight, idx):
    mesh = plsc.VectorSubcoreMesh(core_axis_name="c", subcore_axis_name="s")

    @functools.partial(
        pl.kernel,
        mesh=mesh,
        out_type=jax.ShapeDtypeStruct((B, D), jnp.float32),
        scratch_types=[
            pltpu.VMEM((_B_PER_W,), jnp.int32),
            pltpu.VMEM((_B_PER_W, D), jnp.float32),
            pltpu.SemaphoreType.DMA,
        ],
    )
    def gather_kernel(table_hbm, idx_hbm, out_hbm, idx_v, rows_v, sem):
        wid = lax.axis_index("s") * _NC + lax.axis_index("c")
        base = wid * _B_PER_W
        pltpu.sync_copy(idx_hbm.at[pl.ds(base, _B_PER_W)], idx_v)
        pltpu.async_copy(table_hbm.at[idx_v], rows_v, sem).wait()
        pltpu.sync_copy(rows_v, out_hbm.at[pl.ds(base, _B_PER_W)])

    return gather_kernel(weight, idx)


_BBLK = 128


def _scale_mul_body(rows_ref, x_ref, o_ref):
    i = pl.program_id(0)
    w = rows_ref[pl.ds(i * _BBLK, _BBLK), :]
    s = 2.0 * jax.nn.sigmoid(w)
    o_ref[...] = x_ref[...] * s[:, None, :]


def _tc_scale_mul(rows, x):
    return pl.pallas_call(
        _scale_mul_body,
        grid=(B // _BBLK,),
        in_specs=[
            pl.BlockSpec((B, D), lambda i: (0, 0)),
            pl.BlockSpec((_BBLK, T, D), lambda i: (i, 0, 0)),
        ],
        out_specs=pl.BlockSpec((_BBLK, T, D), lambda i: (i, 0, 0)),
        out_shape=jax.ShapeDtypeStruct((B, T, D), jnp.float32),
    )(rows, x)


def kernel(x, spk_id, weight):
    idx = spk_id.reshape(-1)
    rows = _sc_gather(weight, idx)
    return _tc_scale_mul(rows, x)


# speedup vs baseline: 1.0537x; 1.0150x over previous
import functools

import jax
import jax.numpy as jnp
from jax import lax
from jax.experimental import pallas as pl
from jax.experimental.pallas import tpu as pltpu
from jax.experimental.pallas import tpu_sc as plsc

B, T, D = 1024, 200, 128
V = 100000

_NC, _NS = 2, 16
_NW = _NC * _NS
_B_PER_W = B // _NW


def _sc_gather(weight, idx, num_cores=1):
    mesh = plsc.VectorSubcoreMesh(
        core_axis_name="c", subcore_axis_name="s", num_cores=num_cores
    )
    nw = num_cores * _NS
    b_per_w = B // nw

    @functools.partial(
        pl.kernel,
        mesh=mesh,
        out_type=jax.ShapeDtypeStruct((B, D), jnp.float32),
        scratch_types=[
            pltpu.VMEM((b_per_w,), jnp.int32),
            pltpu.VMEM((b_per_w, D), jnp.float32),
            pltpu.SemaphoreType.DMA,
        ],
    )
    def gather_kernel(table_hbm, idx_hbm, out_hbm, idx_v, rows_v, sem):
        wid = lax.axis_index("s") * num_cores + lax.axis_index("c")
        base = wid * b_per_w
        pltpu.sync_copy(idx_hbm.at[pl.ds(base, b_per_w)], idx_v)
        pltpu.async_copy(table_hbm.at[idx_v], rows_v, sem).wait()
        pltpu.sync_copy(rows_v, out_hbm.at[pl.ds(base, b_per_w)])

    return gather_kernel(weight, idx)


_BBLK = 128


def _scale_mul_body(rows_ref, x_ref, o_ref):
    i = pl.program_id(0)
    w = rows_ref[pl.ds(i * _BBLK, _BBLK), :]
    s = 2.0 * jax.nn.sigmoid(w)
    o_ref[...] = x_ref[...] * s[:, None, :]


def _tc_scale_mul(rows, x):
    return pl.pallas_call(
        _scale_mul_body,
        grid=(B // _BBLK,),
        in_specs=[
            pl.BlockSpec((B, D), lambda i: (0, 0)),
            pl.BlockSpec((_BBLK, T, D), lambda i: (i, 0, 0)),
        ],
        out_specs=pl.BlockSpec((_BBLK, T, D), lambda i: (i, 0, 0)),
        out_shape=jax.ShapeDtypeStruct((B, T, D), jnp.float32),
    )(rows, x)


def kernel(x, spk_id, weight):
    idx = spk_id.reshape(-1)
    rows = _sc_gather(weight, idx)
    return _tc_scale_mul(rows, x)
